# baseline (speedup 1.0000x reference)
import jax
import jax.numpy as jnp
from jax import lax
from jax.experimental import pallas as pl
from jax.experimental.pallas import tpu as pltpu
from jax.experimental.pallas import tpu_sc as plsc

N_NODES = 10000
D = 128
N_EDGES = 320000

NC = 2
NS = 16
NW = NC * NS

K = 128
NCHUNK = 80
EPW = K * NCHUNK
NE_PAD = NW * EPW
PAD = NE_PAD - N_EDGES
PAD_ROWS = 64
NP = N_NODES + PAD_ROWS

NWB = 10
ROWS_PT = N_NODES // NWB
DEG_PT = 640
DEGN = NS * DEG_PT


def _mesh():
    return plsc.VectorSubcoreMesh(
        core_axis_name="c", subcore_axis_name="s", num_cores=NC, num_subcores=NS
    )


def _deg_body(srcw, dstw, degp, acc_s, acc_d, idx_s, idx_d, ones_v, zb, semd):
    c = lax.axis_index("c")
    s = lax.axis_index("s")
    wid = c * NS + s
    zero16 = jnp.zeros((16,), jnp.float32)
    one16 = jnp.full((16,), 1.0, jnp.float32)

    def z(i, carry):
        zb[pl.ds(i * 16, 16)] = zero16
        return carry

    lax.fori_loop(0, DEG_PT // 16, z, 0)

    def o(i, carry):
        ones_v[pl.ds(i * 16, 16)] = one16
        return carry

    lax.fori_loop(0, K // 16, o, 0)

    pltpu.sync_copy(zb, acc_s.at[pl.ds(s * DEG_PT, DEG_PT)])
    pltpu.sync_copy(zb, acc_d.at[pl.ds(s * DEG_PT, DEG_PT)])
    plsc.subcore_barrier()

    pltpu.sync_copy(srcw.at[wid], idx_s)
    pltpu.sync_copy(dstw.at[wid], idx_d)

    GB = 8

    def _issue(g):
        def f(jj, carry):
            j = g * GB + jj
            pltpu.async_copy(ones_v, acc_s.at[idx_s.at[j]], semd, add=True)
            pltpu.async_copy(ones_v, acc_d.at[idx_d.at[j]], semd, add=True)
            return carry
        lax.fori_loop(0, GB, f, 0)

    def _drain(g):
        def f(jj, carry):
            j = g * GB + jj
            pltpu.make_async_copy(ones_v, acc_s.at[idx_s.at[j]], semd).wait()
            pltpu.make_async_copy(ones_v, acc_d.at[idx_d.at[j]], semd).wait()
            return carry
        lax.fori_loop(0, GB, f, 0)

    _issue(0)

    def outer(g, carry):
        @pl.when(g + 1 < NCHUNK // GB)
        def _i():
            _issue(g + 1)

        _drain(g)
        return carry

    lax.fori_loop(0, NCHUNK // GB, outer, 0)
    plsc.subcore_barrier()

    pltpu.sync_copy(acc_s.at[pl.ds(s * DEG_PT, DEG_PT)],
                    degp.at[c, 0, pl.ds(s * DEG_PT, DEG_PT)])
    pltpu.sync_copy(acc_d.at[pl.ds(s * DEG_PT, DEG_PT)],
                    degp.at[c, 1, pl.ds(s * DEG_PT, DEG_PT)])


def _agg_body(h, srcw, dstw, outp, acc, idx_s, idx_d, rows0, rows1,
              sem0, sem1):
    c = lax.axis_index("c")
    s = lax.axis_index("s")
    wid = c * NS + s
    zero16 = jnp.zeros((16,), jnp.float32)

    def z(i, carry):
        rows0[i // 8, pl.ds((i % 8) * 16, 16)] = zero16
        return carry

    lax.fori_loop(0, K * 8, z, 0)

    @pl.when(s < NWB)
    def _zero():
        for kk in range(ROWS_PT // K):
            pltpu.sync_copy(rows0, acc.at[pl.ds(s * ROWS_PT + kk * K, K)])
        rem = ROWS_PT % K
        pltpu.sync_copy(rows0.at[pl.ds(0, rem)],
                        acc.at[pl.ds(s * ROWS_PT + ROWS_PT - rem, rem)])

    @pl.when(s == 0)
    def _zero_pad():
        pltpu.sync_copy(rows0.at[pl.ds(0, PAD_ROWS)],
                        acc.at[pl.ds(N_NODES, PAD_ROWS)])

    plsc.subcore_barrier()

    pltpu.sync_copy(srcw.at[wid], idx_s)
    pltpu.sync_copy(dstw.at[wid, pl.ds(0, NCHUNK // 2)], idx_d)

    rows = (rows0, rows1)
    sems = (sem0, sem1)
    H2 = K // 2

    def _gissue(j, b):
        pltpu.async_copy(h.at[idx_s.at[j, pl.ds(0, H2)]],
                         rows[b].at[pl.ds(0, H2)], sems[b])
        pltpu.async_copy(h.at[idx_s.at[j, pl.ds(H2, H2)]],
                         rows[b].at[pl.ds(H2, H2)], sems[b])

    def _gwait(j, b):
        pltpu.make_async_copy(h.at[idx_s.at[j, pl.ds(0, H2)]],
                              rows[b].at[pl.ds(0, H2)], sems[b]).wait()
        pltpu.make_async_copy(h.at[idx_s.at[j, pl.ds(H2, H2)]],
                              rows[b].at[pl.ds(H2, H2)], sems[b]).wait()

    _gissue(0, 0)
    _gissue(1, 1)

    def make_outer(half):
        def outer(i, carry):
            for b in range(2):
                j = half * (NCHUNK // 2) + 2 * i + b
                _gwait(j, b)
                pltpu.sync_copy(rows[b],
                                acc.at[idx_d.at[j - half * (NCHUNK // 2)]],
                                add=True)

                @pl.when(j + 2 < NCHUNK)
                def _issue():
                    _gissue(j + 2, b)

            return carry
        return outer

    lax.fori_loop(0, NCHUNK // 4, make_outer(0), 0)
    pltpu.sync_copy(dstw.at[wid, pl.ds(NCHUNK // 2, NCHUNK // 2)], idx_d)
    lax.fori_loop(0, NCHUNK // 4, make_outer(1), 0)
    plsc.subcore_barrier()

    @pl.when(s < NWB)
    def _writeback():
        pltpu.sync_copy(acc.at[pl.ds(s * ROWS_PT, ROWS_PT)],
                        outp.at[c, pl.ds(s * ROWS_PT, ROWS_PT)])


def _deg_call(srcw, dstw):
    return pl.kernel(
        _deg_body,
        out_type=jax.ShapeDtypeStruct((NC, 2, DEGN), jnp.float32),
        mesh=_mesh(),
        scratch_types=[
            pltpu.VMEM_SHARED((DEGN,), jnp.float32),
            pltpu.VMEM_SHARED((DEGN,), jnp.float32),
            pltpu.VMEM((NCHUNK, K), jnp.int32),
            pltpu.VMEM((NCHUNK, K), jnp.int32),
            pltpu.VMEM((K,), jnp.float32),
            pltpu.VMEM((DEG_PT,), jnp.float32),
            pltpu.SemaphoreType.DMA,
        ],
    )(srcw, dstw)


def _agg_call(h, srcw, dstw):
    return pl.kernel(
        _agg_body,
        out_type=jax.ShapeDtypeStruct((NC, N_NODES, D), jnp.float32),
        mesh=_mesh(),
        scratch_types=[
            pltpu.VMEM_SHARED((NP, D), jnp.float32),
            pltpu.VMEM((NCHUNK, K), jnp.int32),
            pltpu.VMEM((NCHUNK // 2, K), jnp.int32),
            pltpu.VMEM((K, D), jnp.float32),
            pltpu.VMEM((K, D), jnp.float32),
            pltpu.SemaphoreType.DMA,
            pltpu.SemaphoreType.DMA,
        ],
    )(h, srcw, dstw)


def _norm(d0, d1):
    deg = d0 + d1
    return jnp.where(deg > 0, lax.rsqrt(jnp.maximum(deg, 1e-12)), 0.0)


BR = 1000


def _tcscale_body(x_ref, w_ref, ds0_ref, ds1_ref, h_ref):
    ns = _norm(ds0_ref[...], ds1_ref[...])
    xw = jnp.dot(x_ref[...], w_ref[...], preferred_element_type=jnp.float32)
    h_ref[...] = xw * ns


def _tc2_body(p_ref, b1_ref, w_ref, dd0_ref, dd1_ref, ds0_ref, ds1_ref, h_ref):
    nd = _norm(dd0_ref[...], dd1_ref[...])
    ns = _norm(ds0_ref[...], ds1_ref[...])
    o1 = jnp.maximum((p_ref[0] + p_ref[1]) * nd + b1_ref[...], 0.0)
    h_ref[...] = jnp.dot(o1, w_ref[...], preferred_element_type=jnp.float32) * ns


def _tc3_body(p_ref, b2_ref, dd0_ref, dd1_ref, o_ref):
    nd = _norm(dd0_ref[...], dd1_ref[...])
    o_ref[...] = (p_ref[0] + p_ref[1]) * nd + b2_ref[...]


_ROWS = pl.BlockSpec((BR, D), lambda i: (i, 0))
_NORM = pl.BlockSpec((BR, 1), lambda i: (i, 0))
_WMAT = pl.BlockSpec((D, D), lambda i: (0, 0))
_BVEC = pl.BlockSpec((1, D), lambda i: (0, 0))
_PART = pl.BlockSpec((NC, BR, D), lambda i: (0, i, 0))
_GRID = N_NODES // BR


def _tcscale(x, W1, ds0, ds1):
    return pl.pallas_call(
        _tcscale_body,
        grid=(_GRID,),
        in_specs=[_ROWS, _WMAT, _NORM, _NORM],
        out_specs=_ROWS,
        out_shape=jax.ShapeDtypeStruct((N_NODES, D), jnp.float32),
    )(x, W1, ds0, ds1)


def _tc2(p, b1, W2, dd0, dd1, ds0, ds1):
    return pl.pallas_call(
        _tc2_body,
        grid=(_GRID,),
        in_specs=[_PART, _BVEC, _WMAT, _NORM, _NORM, _NORM, _NORM],
        out_specs=_ROWS,
        out_shape=jax.ShapeDtypeStruct((N_NODES, D), jnp.float32),
    )(p, b1, W2, dd0, dd1, ds0, ds1)


def _tc3(p, b2, dd0, dd1):
    return pl.pallas_call(
        _tc3_body,
        grid=(_GRID,),
        in_specs=[_PART, _BVEC, _NORM, _NORM],
        out_specs=_ROWS,
        out_shape=jax.ShapeDtypeStruct((N_NODES, D), jnp.float32),
    )(p, b2, dd0, dd1)


def kernel(x, edge_index, W1, b1, W2, b2):
    ar = jnp.arange(PAD, dtype=jnp.int32)
    dummy_hi = N_NODES + (ar % PAD_ROWS)
    src_flat = edge_index[0].astype(jnp.int32)
    dst_flat = edge_index[1].astype(jnp.int32)
    src = jnp.concatenate([src_flat, ar % N_NODES]).reshape(NW, NCHUNK, K)
    src_deg = jnp.concatenate([src_flat, dummy_hi]).reshape(NW, NCHUNK, K)
    dst = jnp.concatenate([dst_flat, dummy_hi]).reshape(NW, NCHUNK, K)

    degp = _deg_call(src_deg, dst)
    ds0 = degp[0, 0].reshape(DEGN, 1)
    ds1 = degp[1, 0].reshape(DEGN, 1)
    dd0 = degp[0, 1].reshape(DEGN, 1)
    dd1 = degp[1, 1].reshape(DEGN, 1)

    h1 = _tcscale(x, W1, ds0, ds1)
    p1 = _agg_call(h1, src, dst)
    h2 = _tc2(p1, b1.reshape(1, D), W2, dd0, dd1, ds0, ds1)
    p2 = _agg_call(h2, src, dst)
    return _tc3(p2, b2.reshape(1, D), dd0, dd1)


# speedup vs baseline: 1.0270x; 1.0270x over previous
import jax
import jax.numpy as jnp
from jax import lax
from jax.experimental import pallas as pl
from jax.experimental.pallas import tpu as pltpu
from jax.experimental.pallas import tpu_sc as plsc

N_NODES = 10000
D = 128
N_EDGES = 320000

NC = 2
NS = 16
NW = NC * NS

K = 128
NCHUNK = 80
EPW = K * NCHUNK
NE_PAD = NW * EPW
PAD = NE_PAD - N_EDGES
PAD_ROWS = 64
NP = N_NODES + PAD_ROWS

NWB = 10
ROWS_PT = N_NODES // NWB
DEG_PT = 640
DEGN = NS * DEG_PT


def _mesh():
    return plsc.VectorSubcoreMesh(
        core_axis_name="c", subcore_axis_name="s", num_cores=NC, num_subcores=NS
    )


def _deg_body(srcw, dstw, degp, acc_s, acc_d, idx_s, idx_d, ones_v, zb, semd):
    c = lax.axis_index("c")
    s = lax.axis_index("s")
    wid = c * NS + s
    zero16 = jnp.zeros((16,), jnp.float32)
    one16 = jnp.full((16,), 1.0, jnp.float32)

    def z(i, carry):
        zb[pl.ds(i * 16, 16)] = zero16
        return carry

    lax.fori_loop(0, DEG_PT // 16, z, 0)

    def o(i, carry):
        ones_v[pl.ds(i * 16, 16)] = one16
        return carry

    lax.fori_loop(0, K // 16, o, 0)

    pltpu.sync_copy(zb, acc_s.at[pl.ds(s * DEG_PT, DEG_PT)])
    pltpu.sync_copy(zb, acc_d.at[pl.ds(s * DEG_PT, DEG_PT)])
    plsc.subcore_barrier()

    pltpu.sync_copy(srcw.at[wid], idx_s)
    pltpu.sync_copy(dstw.at[wid], idx_d)

    GB = 8

    def _issue(g):
        def f(jj, carry):
            j = g * GB + jj
            pltpu.async_copy(ones_v, acc_s.at[idx_s.at[j]], semd, add=True)
            pltpu.async_copy(ones_v, acc_d.at[idx_d.at[j]], semd, add=True)
            return carry
        lax.fori_loop(0, GB, f, 0)

    def _drain(g):
        def f(jj, carry):
            j = g * GB + jj
            pltpu.make_async_copy(ones_v, acc_s.at[idx_s.at[j]], semd).wait()
            pltpu.make_async_copy(ones_v, acc_d.at[idx_d.at[j]], semd).wait()
            return carry
        lax.fori_loop(0, GB, f, 0)

    _issue(0)

    def outer(g, carry):
        @pl.when(g + 1 < NCHUNK // GB)
        def _i():
            _issue(g + 1)

        _drain(g)
        return carry

    lax.fori_loop(0, NCHUNK // GB, outer, 0)
    plsc.subcore_barrier()

    pltpu.sync_copy(acc_s.at[pl.ds(s * DEG_PT, DEG_PT)],
                    degp.at[c, 0, pl.ds(s * DEG_PT, DEG_PT)])
    pltpu.sync_copy(acc_d.at[pl.ds(s * DEG_PT, DEG_PT)],
                    degp.at[c, 1, pl.ds(s * DEG_PT, DEG_PT)])


def _agg_body(h, srcw, dstw, outp, acc, idx_s, idx_d, rows0, rows1,
              sem0, sem1):
    c = lax.axis_index("c")
    s = lax.axis_index("s")
    wid = c * NS + s
    zero16 = jnp.zeros((16,), jnp.float32)

    def z(i, carry):
        rows0[i // 8, pl.ds((i % 8) * 16, 16)] = zero16
        return carry

    lax.fori_loop(0, K * 8, z, 0)

    pltpu.async_copy(srcw.at[wid], idx_s, sem1)
    pltpu.async_copy(dstw.at[wid, pl.ds(0, NCHUNK // 2)], idx_d, sem1)

    @pl.when(s < NWB)
    def _zero():
        for kk in range(ROWS_PT // K):
            pltpu.async_copy(rows0, acc.at[pl.ds(s * ROWS_PT + kk * K, K)],
                             sem0)
        rem = ROWS_PT % K
        pltpu.async_copy(rows0.at[pl.ds(0, rem)],
                         acc.at[pl.ds(s * ROWS_PT + ROWS_PT - rem, rem)], sem0)

    @pl.when(s == 0)
    def _zero_pad():
        pltpu.async_copy(rows0.at[pl.ds(0, PAD_ROWS)],
                         acc.at[pl.ds(N_NODES, PAD_ROWS)], sem0)

    @pl.when(s < NWB)
    def _zero_drain():
        for kk in range(ROWS_PT // K):
            pltpu.make_async_copy(
                rows0, acc.at[pl.ds(s * ROWS_PT + kk * K, K)], sem0).wait()
        rem = ROWS_PT % K
        pltpu.make_async_copy(
            rows0.at[pl.ds(0, rem)],
            acc.at[pl.ds(s * ROWS_PT + ROWS_PT - rem, rem)], sem0).wait()

    @pl.when(s == 0)
    def _zero_pad_drain():
        pltpu.make_async_copy(rows0.at[pl.ds(0, PAD_ROWS)],
                              acc.at[pl.ds(N_NODES, PAD_ROWS)], sem0).wait()

    pltpu.make_async_copy(srcw.at[wid], idx_s, sem1).wait()
    pltpu.make_async_copy(dstw.at[wid, pl.ds(0, NCHUNK // 2)],
                          idx_d, sem1).wait()
    plsc.subcore_barrier()

    rows = (rows0, rows1)
    sems = (sem0, sem1)
    H2 = K // 2

    def _gissue(j, b):
        pltpu.async_copy(h.at[idx_s.at[j, pl.ds(0, H2)]],
                         rows[b].at[pl.ds(0, H2)], sems[b])
        pltpu.async_copy(h.at[idx_s.at[j, pl.ds(H2, H2)]],
                         rows[b].at[pl.ds(H2, H2)], sems[b])

    def _gwait(j, b):
        pltpu.make_async_copy(h.at[idx_s.at[j, pl.ds(0, H2)]],
                              rows[b].at[pl.ds(0, H2)], sems[b]).wait()
        pltpu.make_async_copy(h.at[idx_s.at[j, pl.ds(H2, H2)]],
                              rows[b].at[pl.ds(H2, H2)], sems[b]).wait()

    _gissue(0, 0)
    _gissue(1, 1)

    def make_outer(half):
        def outer(i, carry):
            for b in range(2):
                j = half * (NCHUNK // 2) + 2 * i + b
                _gwait(j, b)
                pltpu.sync_copy(rows[b],
                                acc.at[idx_d.at[j - half * (NCHUNK // 2)]],
                                add=True)

                @pl.when(j + 2 < NCHUNK)
                def _issue():
                    _gissue(j + 2, b)

            return carry
        return outer

    lax.fori_loop(0, NCHUNK // 4, make_outer(0), 0)
    pltpu.sync_copy(dstw.at[wid, pl.ds(NCHUNK // 2, NCHUNK // 2)], idx_d)
    lax.fori_loop(0, NCHUNK // 4, make_outer(1), 0)
    plsc.subcore_barrier()

    @pl.when(s < NWB)
    def _writeback():
        pltpu.sync_copy(acc.at[pl.ds(s * ROWS_PT, ROWS_PT)],
                        outp.at[c, pl.ds(s * ROWS_PT, ROWS_PT)])


def _deg_call(srcw, dstw):
    return pl.kernel(
        _deg_body,
        out_type=jax.ShapeDtypeStruct((NC, 2, DEGN), jnp.float32),
        mesh=_mesh(),
        scratch_types=[
            pltpu.VMEM_SHARED((DEGN,), jnp.float32),
            pltpu.VMEM_SHARED((DEGN,), jnp.float32),
            pltpu.VMEM((NCHUNK, K), jnp.int32),
            pltpu.VMEM((NCHUNK, K), jnp.int32),
            pltpu.VMEM((K,), jnp.float32),
            pltpu.VMEM((DEG_PT,), jnp.float32),
            pltpu.SemaphoreType.DMA,
        ],
    )(srcw, dstw)


def _agg_call(h, srcw, dstw):
    return pl.kernel(
        _agg_body,
        out_type=jax.ShapeDtypeStruct((NC, N_NODES, D), jnp.float32),
        mesh=_mesh(),
        scratch_types=[
            pltpu.VMEM_SHARED((NP, D), jnp.float32),
            pltpu.VMEM((NCHUNK, K), jnp.int32),
            pltpu.VMEM((NCHUNK // 2, K), jnp.int32),
            pltpu.VMEM((K, D), jnp.float32),
            pltpu.VMEM((K, D), jnp.float32),
            pltpu.SemaphoreType.DMA,
            pltpu.SemaphoreType.DMA,
        ],
    )(h, srcw, dstw)


def _norm(d0, d1):
    deg = d0 + d1
    return jnp.where(deg > 0, lax.rsqrt(jnp.maximum(deg, 1e-12)), 0.0)


def _tcscale_body(x_ref, w_ref, ds0_ref, ds1_ref, h_ref):
    ns = _norm(ds0_ref[...], ds1_ref[...])[:N_NODES]
    xw = jnp.dot(x_ref[...], w_ref[...], preferred_element_type=jnp.float32)
    h_ref[...] = xw * ns


def _tc2_body(p_ref, b1_ref, w_ref, dd0_ref, dd1_ref, ds0_ref, ds1_ref, h_ref):
    nd = _norm(dd0_ref[...], dd1_ref[...])[:N_NODES]
    ns = _norm(ds0_ref[...], ds1_ref[...])[:N_NODES]
    o1 = jnp.maximum((p_ref[0] + p_ref[1]) * nd + b1_ref[...], 0.0)
    h_ref[...] = jnp.dot(o1, w_ref[...], preferred_element_type=jnp.float32) * ns


def _tc3_body(p_ref, b2_ref, dd0_ref, dd1_ref, o_ref):
    nd = _norm(dd0_ref[...], dd1_ref[...])[:N_NODES]
    o_ref[...] = (p_ref[0] + p_ref[1]) * nd + b2_ref[...]


def _tcscale(x, W1, ds0, ds1):
    return pl.pallas_call(
        _tcscale_body,
        out_shape=jax.ShapeDtypeStruct((N_NODES, D), jnp.float32),
    )(x, W1, ds0, ds1)


def _tc2(p, b1, W2, dd0, dd1, ds0, ds1):
    return pl.pallas_call(
        _tc2_body,
        out_shape=jax.ShapeDtypeStruct((N_NODES, D), jnp.float32),
    )(p, b1, W2, dd0, dd1, ds0, ds1)


def _tc3(p, b2, dd0, dd1):
    return pl.pallas_call(
        _tc3_body,
        out_shape=jax.ShapeDtypeStruct((N_NODES, D), jnp.float32),
    )(p, b2, dd0, dd1)


def kernel(x, edge_index, W1, b1, W2, b2):
    ar = jnp.arange(PAD, dtype=jnp.int32)
    dummy_hi = N_NODES + (ar % PAD_ROWS)
    src_flat = edge_index[0].astype(jnp.int32)
    dst_flat = edge_index[1].astype(jnp.int32)
    src = jnp.concatenate([src_flat, ar % N_NODES]).reshape(NW, NCHUNK, K)
    src_deg = jnp.concatenate([src_flat, dummy_hi]).reshape(NW, NCHUNK, K)
    dst = jnp.concatenate([dst_flat, dummy_hi]).reshape(NW, NCHUNK, K)

    degp = _deg_call(src_deg, dst)
    ds0 = degp[0, 0].reshape(DEGN, 1)
    ds1 = degp[1, 0].reshape(DEGN, 1)
    dd0 = degp[0, 1].reshape(DEGN, 1)
    dd1 = degp[1, 1].reshape(DEGN, 1)

    h1 = _tcscale(x, W1, ds0, ds1)
    p1 = _agg_call(h1, src, dst)
    h2 = _tc2(p1, b1.reshape(1, D), W2, dd0, dd1, ds0, ds1)
    p2 = _agg_call(h2, src, dst)
    return _tc3(p2, b2.reshape(1, D), dd0, dd1)


# speedup vs baseline: 1.0408x; 1.0135x over previous
import jax
import jax.numpy as jnp
from jax import lax
from jax.experimental import pallas as pl
from jax.experimental.pallas import tpu as pltpu
from jax.experimental.pallas import tpu_sc as plsc

N_NODES = 10000
D = 128
N_EDGES = 320000

NC = 2
NS = 16
NW = NC * NS

K = 128
NCHUNK = 80
EPW = K * NCHUNK
NE_PAD = NW * EPW
PAD = NE_PAD - N_EDGES
PAD_ROWS = 64
NP = N_NODES + PAD_ROWS

WB = 624
WB_TAIL = NP - NS * WB
DEG_PT = 640
DEGN = NS * DEG_PT


def _mesh():
    return plsc.VectorSubcoreMesh(
        core_axis_name="c", subcore_axis_name="s", num_cores=NC, num_subcores=NS
    )


def _deg_body(srcw, dstw, degp, acc_s, acc_d, idx_s, idx_d, ones_v, zb, semd):
    c = lax.axis_index("c")
    s = lax.axis_index("s")
    wid = c * NS + s
    zero16 = jnp.zeros((16,), jnp.float32)
    one16 = jnp.full((16,), 1.0, jnp.float32)

    def z(i, carry):
        zb[pl.ds(i * 16, 16)] = zero16
        return carry

    lax.fori_loop(0, DEG_PT // 16, z, 0)

    def o(i, carry):
        ones_v[pl.ds(i * 16, 16)] = one16
        return carry

    lax.fori_loop(0, K // 16, o, 0)

    pltpu.sync_copy(zb, acc_s.at[pl.ds(s * DEG_PT, DEG_PT)])
    pltpu.sync_copy(zb, acc_d.at[pl.ds(s * DEG_PT, DEG_PT)])
    plsc.subcore_barrier()

    pltpu.sync_copy(srcw.at[wid], idx_s)
    pltpu.sync_copy(dstw.at[wid], idx_d)

    GB = 8

    def _issue(g):
        def f(jj, carry):
            j = g * GB + jj
            pltpu.async_copy(ones_v, acc_s.at[idx_s.at[j]], semd, add=True)
            pltpu.async_copy(ones_v, acc_d.at[idx_d.at[j]], semd, add=True)
            return carry
        lax.fori_loop(0, GB, f, 0)

    def _drain(g):
        def f(jj, carry):
            j = g * GB + jj
            pltpu.make_async_copy(ones_v, acc_s.at[idx_s.at[j]], semd).wait()
            pltpu.make_async_copy(ones_v, acc_d.at[idx_d.at[j]], semd).wait()
            return carry
        lax.fori_loop(0, GB, f, 0)

    _issue(0)

    def outer(g, carry):
        @pl.when(g + 1 < NCHUNK // GB)
        def _i():
            _issue(g + 1)

        _drain(g)
        return carry

    lax.fori_loop(0, NCHUNK // GB, outer, 0)
    plsc.subcore_barrier()

    pltpu.sync_copy(acc_s.at[pl.ds(s * DEG_PT, DEG_PT)],
                    degp.at[c, 0, pl.ds(s * DEG_PT, DEG_PT)])
    pltpu.sync_copy(acc_d.at[pl.ds(s * DEG_PT, DEG_PT)],
                    degp.at[c, 1, pl.ds(s * DEG_PT, DEG_PT)])


def _agg_body(h, srcw, dstw, outp, acc, idx_s, idx_d, rows0, rows1,
              sem0, sem1):
    c = lax.axis_index("c")
    s = lax.axis_index("s")
    wid = c * NS + s
    zero16 = jnp.zeros((16,), jnp.float32)

    def z(i, carry):
        rows0[i // 8, pl.ds((i % 8) * 16, 16)] = zero16
        return carry

    lax.fori_loop(0, K * 8, z, 0)

    pltpu.async_copy(srcw.at[wid], idx_s, sem1)
    pltpu.async_copy(dstw.at[wid, pl.ds(0, NCHUNK // 2)], idx_d, sem1)

    for kk in range(WB // K):
        pltpu.async_copy(rows0, acc.at[pl.ds(s * WB + kk * K, K)], sem0)
    rem = WB % K
    pltpu.async_copy(rows0.at[pl.ds(0, rem)],
                     acc.at[pl.ds(s * WB + WB - rem, rem)], sem0)

    @pl.when(s == 0)
    def _zero_tail():
        pltpu.async_copy(rows0.at[pl.ds(0, WB_TAIL)],
                         acc.at[pl.ds(NS * WB, WB_TAIL)], sem0)

    for kk in range(WB // K):
        pltpu.make_async_copy(rows0, acc.at[pl.ds(s * WB + kk * K, K)],
                              sem0).wait()
    pltpu.make_async_copy(rows0.at[pl.ds(0, rem)],
                          acc.at[pl.ds(s * WB + WB - rem, rem)], sem0).wait()

    @pl.when(s == 0)
    def _zero_tail_drain():
        pltpu.make_async_copy(rows0.at[pl.ds(0, WB_TAIL)],
                              acc.at[pl.ds(NS * WB, WB_TAIL)], sem0).wait()

    pltpu.make_async_copy(srcw.at[wid], idx_s, sem1).wait()
    pltpu.make_async_copy(dstw.at[wid, pl.ds(0, NCHUNK // 2)],
                          idx_d, sem1).wait()
    plsc.subcore_barrier()

    rows = (rows0, rows1)
    sems = (sem0, sem1)
    H2 = K // 2

    def _gissue(j, b):
        pltpu.async_copy(h.at[idx_s.at[j, pl.ds(0, H2)]],
                         rows[b].at[pl.ds(0, H2)], sems[b])
        pltpu.async_copy(h.at[idx_s.at[j, pl.ds(H2, H2)]],
                         rows[b].at[pl.ds(H2, H2)], sems[b])

    def _gwait(j, b):
        pltpu.make_async_copy(h.at[idx_s.at[j, pl.ds(0, H2)]],
                              rows[b].at[pl.ds(0, H2)], sems[b]).wait()
        pltpu.make_async_copy(h.at[idx_s.at[j, pl.ds(H2, H2)]],
                              rows[b].at[pl.ds(H2, H2)], sems[b]).wait()

    _gissue(0, 0)
    _gissue(1, 1)

    def make_outer(half):
        def outer(i, carry):
            for b in range(2):
                j = half * (NCHUNK // 2) + 2 * i + b
                _gwait(j, b)
                pltpu.sync_copy(rows[b],
                                acc.at[idx_d.at[j - half * (NCHUNK // 2)]],
                                add=True)

                @pl.when(j + 2 < NCHUNK)
                def _issue():
                    _gissue(j + 2, b)

            return carry
        return outer

    lax.fori_loop(0, NCHUNK // 4, make_outer(0), 0)
    pltpu.sync_copy(dstw.at[wid, pl.ds(NCHUNK // 2, NCHUNK // 2)], idx_d)
    lax.fori_loop(0, NCHUNK // 4, make_outer(1), 0)
    plsc.subcore_barrier()

    pltpu.sync_copy(acc.at[pl.ds(s * WB, WB)], outp.at[c, pl.ds(s * WB, WB)])

    @pl.when(s == 0)
    def _writeback_tail():
        nreal = N_NODES - NS * WB
        pltpu.sync_copy(acc.at[pl.ds(NS * WB, nreal)],
                        outp.at[c, pl.ds(NS * WB, nreal)])


def _deg_call(srcw, dstw):
    return pl.kernel(
        _deg_body,
        out_type=jax.ShapeDtypeStruct((NC, 2, DEGN), jnp.float32),
        mesh=_mesh(),
        scratch_types=[
            pltpu.VMEM_SHARED((DEGN,), jnp.float32),
            pltpu.VMEM_SHARED((DEGN,), jnp.float32),
            pltpu.VMEM((NCHUNK, K), jnp.int32),
            pltpu.VMEM((NCHUNK, K), jnp.int32),
            pltpu.VMEM((K,), jnp.float32),
            pltpu.VMEM((DEG_PT,), jnp.float32),
            pltpu.SemaphoreType.DMA,
        ],
    )(srcw, dstw)


def _agg_call(h, srcw, dstw):
    return pl.kernel(
        _agg_body,
        out_type=jax.ShapeDtypeStruct((NC, N_NODES, D), jnp.float32),
        mesh=_mesh(),
        scratch_types=[
            pltpu.VMEM_SHARED((NP, D), jnp.float32),
            pltpu.VMEM((NCHUNK, K), jnp.int32),
            pltpu.VMEM((NCHUNK // 2, K), jnp.int32),
            pltpu.VMEM((K, D), jnp.float32),
            pltpu.VMEM((K, D), jnp.float32),
            pltpu.SemaphoreType.DMA,
            pltpu.SemaphoreType.DMA,
        ],
    )(h, srcw, dstw)


def _norm(d0, d1):
    deg = d0 + d1
    return jnp.where(deg > 0, lax.rsqrt(jnp.maximum(deg, 1e-12)), 0.0)


def _tcscale_body(x_ref, w_ref, ds0_ref, ds1_ref, h_ref):
    ns = _norm(ds0_ref[...], ds1_ref[...])[:N_NODES]
    xw = jnp.dot(x_ref[...], w_ref[...], preferred_element_type=jnp.float32)
    h_ref[...] = xw * ns


def _tc2_body(p_ref, b1_ref, w_ref, dd0_ref, dd1_ref, ds0_ref, ds1_ref, h_ref):
    nd = _norm(dd0_ref[...], dd1_ref[...])[:N_NODES]
    ns = _norm(ds0_ref[...], ds1_ref[...])[:N_NODES]
    o1 = jnp.maximum((p_ref[0] + p_ref[1]) * nd + b1_ref[...], 0.0)
    h_ref[...] = jnp.dot(o1, w_ref[...], preferred_element_type=jnp.float32) * ns


def _tc3_body(p_ref, b2_ref, dd0_ref, dd1_ref, o_ref):
    nd = _norm(dd0_ref[...], dd1_ref[...])[:N_NODES]
    o_ref[...] = (p_ref[0] + p_ref[1]) * nd + b2_ref[...]


def _tcscale(x, W1, ds0, ds1):
    return pl.pallas_call(
        _tcscale_body,
        out_shape=jax.ShapeDtypeStruct((N_NODES, D), jnp.float32),
    )(x, W1, ds0, ds1)


def _tc2(p, b1, W2, dd0, dd1, ds0, ds1):
    return pl.pallas_call(
        _tc2_body,
        out_shape=jax.ShapeDtypeStruct((N_NODES, D), jnp.float32),
    )(p, b1, W2, dd0, dd1, ds0, ds1)


def _tc3(p, b2, dd0, dd1):
    return pl.pallas_call(
        _tc3_body,
        out_shape=jax.ShapeDtypeStruct((N_NODES, D), jnp.float32),
    )(p, b2, dd0, dd1)


def kernel(x, edge_index, W1, b1, W2, b2):
    ar = jnp.arange(PAD, dtype=jnp.int32)
    dummy_hi = N_NODES + (ar % PAD_ROWS)
    src_flat = edge_index[0].astype(jnp.int32)
    dst_flat = edge_index[1].astype(jnp.int32)
    src = jnp.concatenate([src_flat, ar % N_NODES]).reshape(NW, NCHUNK, K)
    src_deg = jnp.concatenate([src_flat, dummy_hi]).reshape(NW, NCHUNK, K)
    dst = jnp.concatenate([dst_flat, dummy_hi]).reshape(NW, NCHUNK, K)

    degp = _deg_call(src_deg, dst)
    ds0 = degp[0, 0].reshape(DEGN, 1)
    ds1 = degp[1, 0].reshape(DEGN, 1)
    dd0 = degp[0, 1].reshape(DEGN, 1)
    dd1 = degp[1, 1].reshape(DEGN, 1)

    h1 = _tcscale(x, W1, ds0, ds1)
    p1 = _agg_call(h1, src, dst)
    h2 = _tc2(p1, b1.reshape(1, D), W2, dd0, dd1, ds0, ds1)
    p2 = _agg_call(h2, src, dst)
    return _tc3(p2, b2.reshape(1, D), dd0, dd1)
